# baseline (speedup 1.0000x reference)
import functools
import jax
import jax.numpy as jnp
from jax import lax
from jax.experimental import pallas as pl
from jax.experimental.pallas import tpu as pltpu
from jax.experimental.pallas import tpu_sc as plsc

_OUT = 7
_BINS = _OUT * _OUT
_TPB = 16
_NTAP = _BINS * _TPB
_CHUNK_BINS = 7
_CHUNK_ROWS = _CHUNK_BINS * _TPB
_C = 256
_W32 = _C // 2
_NPAD = 1024
_NROI = 1000
_RPT = 32
_NCHUNKS = _RPT * (_BINS // _CHUNK_BINS)
_LVL_OFF = (0, 2 * 128 * 128, 2 * 128 * 128 + 2 * 64 * 64,
            2 * 128 * 128 + 2 * 64 * 64 + 2 * 32 * 32)
_T1 = float((56.0 * (2.0 - 1e-6)) ** 2)
_T2 = float((56.0 * (4.0 - 1e-6)) ** 2)
_T3 = float((56.0 * (8.0 - 1e-6)) ** 2)

_GATHER_DNUMS = lax.GatherDimensionNumbers(
    offset_dims=(), collapsed_slice_dims=(0,), start_index_map=(0,))


def _dg(v, idx):
    return lax.gather(v, idx[:, None], dimension_numbers=_GATHER_DNUMS,
                      slice_sizes=(1,), mode=lax.GatherScatterMode.PROMISE_IN_BOUNDS)


def _splat_i32(x):
    return jnp.full((16,), x, jnp.int32)


def _sc_body(bidx_h, x1_h, y1_h, x2_h, y2_h, table_h, out_h,
             bidx_v, x1_v, y1_v, x2_v, y2_v,
             idx_b, w_b, g_ring, o_buf, gsem, out_sem):
    wid = lax.axis_index("s") * 2 + lax.axis_index("c")
    tbase = wid * _RPT

    pltpu.sync_copy(bidx_h.at[pl.ds(tbase, _RPT)], bidx_v)
    pltpu.sync_copy(x1_h.at[pl.ds(tbase, _RPT)], x1_v)
    pltpu.sync_copy(y1_h.at[pl.ds(tbase, _RPT)], y1_v)
    pltpu.sync_copy(x2_h.at[pl.ds(tbase, _RPT)], x2_v)
    pltpu.sync_copy(y2_h.at[pl.ds(tbase, _RPT)], y2_v)

    iota = lax.iota(jnp.int32, 16)
    fiota = iota.astype(jnp.float32)
    dy_l = lax.shift_right_logical(iota, 3) & 1
    dx_l = lax.shift_right_logical(iota, 2) & 1
    cy_l = lax.shift_right_logical(iota, 1) & 1
    cx_l = iota & 1
    cy_f = cy_l.astype(jnp.float32)
    cx_f = cx_l.astype(jnp.float32)
    sample_off = (fiota + 0.5) * 0.5
    one = jnp.full((16,), 1, jnp.int32)
    zero = jnp.full((16,), 0, jnp.int32)

    for grp in range(_RPT // 16):
        sl = pl.ds(grp * 16, 16)
        bi = bidx_v[sl].astype(jnp.int32)
        x1 = x1_v[sl]
        y1 = y1_v[sl]
        x2 = x2_v[sl]
        y2 = y2_v[sl]
        a2 = (x2 - x1 + 1.0) * (y2 - y1 + 1.0)
        lvl = (jnp.where(a2 >= _T1, one, zero) + jnp.where(a2 >= _T2, one, zero)
               + jnp.where(a2 >= _T3, one, zero))
        hsz = lax.shift_right_logical(jnp.full((16,), 128, jnp.int32), lvl)
        hw = lax.shift_right_logical(jnp.full((16,), 128 * 128, jnp.int32), 2 * lvl)
        off = jnp.where(lvl == 0, jnp.full((16,), _LVL_OFF[0], jnp.int32),
              jnp.where(lvl == 1, jnp.full((16,), _LVL_OFF[1], jnp.int32),
              jnp.where(lvl == 2, jnp.full((16,), _LVL_OFF[2], jnp.int32),
                        jnp.full((16,), _LVL_OFF[3], jnp.int32))))
        scale = jnp.where(lvl == 0, jnp.full((16,), 0.25, jnp.float32),
                jnp.where(lvl == 1, jnp.full((16,), 0.125, jnp.float32),
                jnp.where(lvl == 2, jnp.full((16,), 0.0625, jnp.float32),
                          jnp.full((16,), 0.03125, jnp.float32))))
        x1s_g = x1 * scale
        y1s_g = y1 * scale
        bw_g = jnp.maximum(x2 * scale - x1s_g, 1.0) / 7.0
        bh_g = jnp.maximum(y2 * scale - y1s_g, 1.0) / 7.0
        base_g = off + bi * hw
        hm1_g = hsz - 1
        hf_g = hsz.astype(jnp.float32)

        def roi_body(r, _, base_g=base_g, hm1_g=hm1_g, hf_g=hf_g, hsz_g=hsz,
                     x1s_g=x1s_g, y1s_g=y1s_g, bw_g=bw_g, bh_g=bh_g, grp=grp):
            rr = _splat_i32(r)
            base_v = _dg(base_g, rr)
            w_v = _dg(hsz_g, rr)
            hm1_v = _dg(hm1_g, rr)
            hf_v = _dg(hf_g, rr)
            x1s_v = _dg(x1s_g, rr)
            y1s_v = _dg(y1s_g, rr)
            bw_v = _dg(bw_g, rr)
            bh_v = _dg(bh_g, rr)
            tap0 = (grp * 16 + r) * _NTAP

            def axis_taps(start, binsz):
                s = start + sample_off * binsz
                m = jnp.where((s >= -1.0) & (s <= hf_v), 1.0, 0.0)
                sc = jnp.minimum(jnp.maximum(s, 0.0), hf_v - 1.0)
                i0 = sc.astype(jnp.int32)
                l = (sc - i0.astype(jnp.float32)) * m
                h = m - l
                return i0, h, l

            x0, hx, lx = axis_taps(x1s_v, bw_v)
            y0, hy, ly = axis_taps(y1s_v, bh_v)

            def bin_body(g, _c):
                oy = g // 7
                ox = g - 7 * oy
                iy = _splat_i32(2 * oy) + dy_l
                ix = _splat_i32(2 * ox) + dx_l
                y0s = _dg(y0, iy)
                hys = _dg(hy, iy)
                lys = _dg(ly, iy)
                x0s = _dg(x0, ix)
                hxs = _dg(hx, ix)
                lxs = _dg(lx, ix)
                wy = hys + cy_f * (lys - hys)
                wx = hxs + cx_f * (lxs - hxs)
                ytap = jnp.minimum(y0s + cy_l, hm1_v)
                xtap = jnp.minimum(x0s + cx_l, hm1_v)
                idx_b[pl.ds(tap0 + g * _TPB, 16)] = base_v + ytap * w_v + xtap
                w_b[pl.ds(tap0 + g * _TPB, 16)] = wy * wx * 0.25
                return _c

            lax.fori_loop(0, _BINS, bin_body, 0)
            return _

        lax.fori_loop(0, 16, roi_body, 0)

    def issue(chunk, slot):
        return pltpu.async_copy(
            table_h.at[idx_b.at[pl.ds(chunk * _CHUNK_ROWS, _CHUNK_ROWS)]],
            g_ring.at[pl.ds(slot * _CHUNK_ROWS, _CHUNK_ROWS)], gsem)

    issue(0, 0)
    issue(1, 1)

    def chunk_body(t, _):
        @pl.when(t + 2 < _NCHUNKS)
        def _issue_next():
            issue(t + 2, lax.rem(t + 2, 3))

        pltpu.make_async_copy(
            table_h.at[idx_b.at[pl.ds(0, _CHUNK_ROWS)]],
            g_ring.at[pl.ds(0, _CHUNK_ROWS)], gsem).wait()

        rowbase = lax.rem(t, 3) * _CHUNK_ROWS
        roi = t // 7
        ck = t - 7 * roi
        obase = (roi & 1) * (_BINS * _C)

        @pl.when((ck == 0) & (roi >= 2) & (tbase + roi - 2 < _NROI))
        def _drain_out():
            pltpu.make_async_copy(
                o_buf.at[pl.ds(0, _BINS * _C)],
                out_h.at[tbase], out_sem).wait()

        def cbin_body(bl, _c, roi=roi, ck=ck, rowbase=rowbase, obase=obase):
            g = ck * _CHUNK_BINS + bl
            wv = w_b[pl.ds((roi * _BINS + g) * _TPB, 16)]
            acc = [jnp.zeros((16,), jnp.float32) for _ in range(2 * (_W32 // 16))]
            for tp in range(_TPB):
                wt = _dg(wv, _splat_i32(tp))
                row = rowbase + bl * _TPB + tp
                for cg in range(_W32 // 16):
                    w32 = g_ring[row, pl.ds(cg * 16, 16)]
                    flo = lax.bitcast_convert_type(
                        lax.shift_left(w32, 16), jnp.float32)
                    fhi = lax.bitcast_convert_type(w32, jnp.float32)
                    acc[2 * cg] = acc[2 * cg] + wt * flo
                    acc[2 * cg + 1] = acc[2 * cg + 1] + wt * fhi
            for cg in range(_W32 // 16):
                o_buf[pl.ds(obase + g * _C + cg * 16, 16)] = acc[2 * cg]
                o_buf[pl.ds(obase + g * _C + 128 + cg * 16, 16)] = acc[2 * cg + 1]
            return _c

        lax.fori_loop(0, _CHUNK_BINS, cbin_body, 0)

        gro = tbase + roi

        @pl.when((ck == 6) & (gro < _NROI))
        def _flush():
            pltpu.async_copy(o_buf.at[pl.ds(obase, _BINS * _C)],
                             out_h.at[gro], out_sem)
        return _

    lax.fori_loop(0, _NCHUNKS, chunk_body, 0)
    n_out = jnp.minimum(jnp.maximum(_NROI - tbase, 0), _RPT)
    n_pending = jnp.maximum(n_out - (_RPT - 2), 0)

    def drain_body(i, _):
        pltpu.make_async_copy(o_buf.at[pl.ds(0, _BINS * _C)],
                              out_h.at[tbase], out_sem).wait()
        return _

    lax.fori_loop(0, n_pending, drain_body, 0)


def _prep_body(x_ref, o_ref):
    x = x_ref[0]
    c, r, w = x.shape
    x2 = x.reshape(c, r * w)
    lo = x2[:128, :].T.astype(jnp.bfloat16)
    hi = x2[128:, :].T.astype(jnp.bfloat16)
    lo32 = lax.bitcast_convert_type(lo, jnp.uint16).astype(jnp.uint32)
    hi32 = lax.bitcast_convert_type(hi, jnp.uint16).astype(jnp.uint32)
    o_ref[...] = lax.bitcast_convert_type(lo32 | (hi32 << 16), jnp.int32)


def _prep_level(f):
    b, c, h, w = f.shape
    return pl.pallas_call(
        _prep_body,
        grid=(b, h // 8),
        in_specs=[pl.BlockSpec((1, c, 8, w), lambda i, j: (i, 0, j, 0))],
        out_specs=pl.BlockSpec((8 * w, c // 2), lambda i, j, h=h: (i * (h // 8) + j, 0)),
        out_shape=jax.ShapeDtypeStruct((b * h * w, c // 2), jnp.int32),
    )(f)


@jax.jit
def _run(bidx, x1, y1, x2, y2, table):
    mesh = plsc.VectorSubcoreMesh(core_axis_name="c", subcore_axis_name="s")
    f = functools.partial(
        pl.kernel,
        out_type=jax.ShapeDtypeStruct((_NROI, _BINS * _C), jnp.float32),
        mesh=mesh,
        scratch_types=[
            pltpu.VMEM((_RPT,), jnp.float32),
            pltpu.VMEM((_RPT,), jnp.float32),
            pltpu.VMEM((_RPT,), jnp.float32),
            pltpu.VMEM((_RPT,), jnp.float32),
            pltpu.VMEM((_RPT,), jnp.float32),
            pltpu.VMEM((_RPT * _NTAP,), jnp.int32),
            pltpu.VMEM((_RPT * _NTAP,), jnp.float32),
            pltpu.VMEM((3 * _CHUNK_ROWS, _W32), jnp.int32),
            pltpu.VMEM((2 * _BINS * _C,), jnp.float32),
            pltpu.SemaphoreType.DMA,
            pltpu.SemaphoreType.DMA,
        ],
    )(_sc_body)
    return f(bidx, x1, y1, x2, y2, table)


def kernel(feat0, feat1, feat2, feat3, rois):
    feats = [feat0, feat1, feat2, feat3]
    c = feat0.shape[1]
    nrows = _LVL_OFF[3] + 2 * 16 * 16
    table = jnp.zeros((nrows, c // 2), jnp.int32)
    for f, off in zip(feats, _LVL_OFF):
        table = lax.dynamic_update_slice(table, _prep_level(f), (off, 0))
    n = rois.shape[0]
    pad = jnp.broadcast_to(
        jnp.array([0.0, 0.0, 0.0, 10.0, 10.0], jnp.float32), (_NPAD - n, 5))
    rp = jnp.concatenate([rois, pad], axis=0)
    out = _run(rp[:, 0], rp[:, 1], rp[:, 2], rp[:, 3], rp[:, 4], table)
    return out.reshape(n, _OUT, _OUT, c).transpose(0, 3, 1, 2)


# speedup vs baseline: 1.1205x; 1.1205x over previous
import functools
import jax
import jax.numpy as jnp
from jax import lax
from jax.experimental import pallas as pl
from jax.experimental.pallas import tpu as pltpu
from jax.experimental.pallas import tpu_sc as plsc

_OUT = 7
_BINS = _OUT * _OUT
_TPB = 16
_NTAP = _BINS * _TPB
_CHUNK_BINS = 7
_CHUNK_ROWS = _CHUNK_BINS * _TPB
_C = 256
_W32 = _C // 2
_NPAD = 1024
_NROI = 1000
_RPT = 32
_NCHUNKS = _RPT * (_BINS // _CHUNK_BINS)
_LVL_OFF = (0, 2 * 128 * 128, 2 * 128 * 128 + 2 * 64 * 64,
            2 * 128 * 128 + 2 * 64 * 64 + 2 * 32 * 32)
_T1 = float((56.0 * (2.0 - 1e-6)) ** 2)
_T2 = float((56.0 * (4.0 - 1e-6)) ** 2)
_T3 = float((56.0 * (8.0 - 1e-6)) ** 2)

_GATHER_DNUMS = lax.GatherDimensionNumbers(
    offset_dims=(), collapsed_slice_dims=(0,), start_index_map=(0,))


def _dg(v, idx):
    return lax.gather(v, idx[:, None], dimension_numbers=_GATHER_DNUMS,
                      slice_sizes=(1,), mode=lax.GatherScatterMode.PROMISE_IN_BOUNDS)


def _splat_i32(x):
    return jnp.full((16,), x, jnp.int32)


def _sc_body(bidx_h, x1_h, y1_h, x2_h, y2_h, table_h, out_h,
             bidx_v, x1_v, y1_v, x2_v, y2_v,
             idx_b, w_b, g_ring, o_buf, gsem, out_sem):
    wid = lax.axis_index("s") * 2 + lax.axis_index("c")
    tbase = wid * _RPT

    pltpu.sync_copy(bidx_h.at[pl.ds(tbase, _RPT)], bidx_v)
    pltpu.sync_copy(x1_h.at[pl.ds(tbase, _RPT)], x1_v)
    pltpu.sync_copy(y1_h.at[pl.ds(tbase, _RPT)], y1_v)
    pltpu.sync_copy(x2_h.at[pl.ds(tbase, _RPT)], x2_v)
    pltpu.sync_copy(y2_h.at[pl.ds(tbase, _RPT)], y2_v)

    iota = lax.iota(jnp.int32, 16)
    fiota = iota.astype(jnp.float32)
    dy_l = lax.shift_right_logical(iota, 3) & 1
    dx_l = lax.shift_right_logical(iota, 2) & 1
    cy_l = lax.shift_right_logical(iota, 1) & 1
    cx_l = iota & 1
    cy_f = cy_l.astype(jnp.float32)
    cx_f = cx_l.astype(jnp.float32)
    sample_off = (fiota + 0.5) * 0.5
    one = jnp.full((16,), 1, jnp.int32)
    zero = jnp.full((16,), 0, jnp.int32)

    for grp in range(_RPT // 16):
        sl = pl.ds(grp * 16, 16)
        bi = bidx_v[sl].astype(jnp.int32)
        x1 = x1_v[sl]
        y1 = y1_v[sl]
        x2 = x2_v[sl]
        y2 = y2_v[sl]
        a2 = (x2 - x1 + 1.0) * (y2 - y1 + 1.0)
        lvl = (jnp.where(a2 >= _T1, one, zero) + jnp.where(a2 >= _T2, one, zero)
               + jnp.where(a2 >= _T3, one, zero))
        hsz = lax.shift_right_logical(jnp.full((16,), 128, jnp.int32), lvl)
        hw = lax.shift_right_logical(jnp.full((16,), 128 * 128, jnp.int32), 2 * lvl)
        off = jnp.where(lvl == 0, jnp.full((16,), _LVL_OFF[0], jnp.int32),
              jnp.where(lvl == 1, jnp.full((16,), _LVL_OFF[1], jnp.int32),
              jnp.where(lvl == 2, jnp.full((16,), _LVL_OFF[2], jnp.int32),
                        jnp.full((16,), _LVL_OFF[3], jnp.int32))))
        scale = jnp.where(lvl == 0, jnp.full((16,), 0.25, jnp.float32),
                jnp.where(lvl == 1, jnp.full((16,), 0.125, jnp.float32),
                jnp.where(lvl == 2, jnp.full((16,), 0.0625, jnp.float32),
                          jnp.full((16,), 0.03125, jnp.float32))))
        x1s_g = x1 * scale
        y1s_g = y1 * scale
        bw_g = jnp.maximum(x2 * scale - x1s_g, 1.0) / 7.0
        bh_g = jnp.maximum(y2 * scale - y1s_g, 1.0) / 7.0
        base_g = off + bi * hw
        hm1_g = hsz - 1
        hf_g = hsz.astype(jnp.float32)

        def roi_body(r, _, base_g=base_g, hm1_g=hm1_g, hf_g=hf_g, hsz_g=hsz,
                     x1s_g=x1s_g, y1s_g=y1s_g, bw_g=bw_g, bh_g=bh_g, grp=grp):
            rr = _splat_i32(r)
            base_v = _dg(base_g, rr)
            w_v = _dg(hsz_g, rr)
            hm1_v = _dg(hm1_g, rr)
            hf_v = _dg(hf_g, rr)
            x1s_v = _dg(x1s_g, rr)
            y1s_v = _dg(y1s_g, rr)
            bw_v = _dg(bw_g, rr)
            bh_v = _dg(bh_g, rr)
            tap0 = (grp * 16 + r) * _NTAP

            def axis_taps(start, binsz):
                s = start + sample_off * binsz
                m = jnp.where((s >= -1.0) & (s <= hf_v), 1.0, 0.0)
                sc = jnp.minimum(jnp.maximum(s, 0.0), hf_v - 1.0)
                i0 = sc.astype(jnp.int32)
                l = (sc - i0.astype(jnp.float32)) * m
                h = m - l
                return i0, h, l

            x0, hx, lx = axis_taps(x1s_v, bw_v)
            y0, hy, ly = axis_taps(y1s_v, bh_v)

            def bin_body(g, _c):
                oy = g // 7
                ox = g - 7 * oy
                iy = _splat_i32(2 * oy) + dy_l
                ix = _splat_i32(2 * ox) + dx_l
                y0s = _dg(y0, iy)
                hys = _dg(hy, iy)
                lys = _dg(ly, iy)
                x0s = _dg(x0, ix)
                hxs = _dg(hx, ix)
                lxs = _dg(lx, ix)
                wy = hys + cy_f * (lys - hys)
                wx = hxs + cx_f * (lxs - hxs)
                ytap = jnp.minimum(y0s + cy_l, hm1_v)
                xtap = jnp.minimum(x0s + cx_l, hm1_v)
                idx_b[pl.ds(tap0 + g * _TPB, 16)] = base_v + ytap * w_v + xtap
                w_b[pl.ds(tap0 + g * _TPB, 16)] = wy * wx * 0.25
                return _c

            lax.fori_loop(0, _BINS, bin_body, 0)
            return _

        lax.fori_loop(0, 16, roi_body, 0)

    def issue(chunk, slot):
        return pltpu.async_copy(
            table_h.at[idx_b.at[pl.ds(chunk * _CHUNK_ROWS, _CHUNK_ROWS)]],
            g_ring.at[pl.ds(slot * _CHUNK_ROWS, _CHUNK_ROWS)], gsem)

    issue(0, 0)
    issue(1, 1)

    def chunk_body(t, _):
        @pl.when(t + 2 < _NCHUNKS)
        def _issue_next():
            issue(t + 2, lax.rem(t + 2, 3))

        pltpu.make_async_copy(
            table_h.at[idx_b.at[pl.ds(0, _CHUNK_ROWS)]],
            g_ring.at[pl.ds(0, _CHUNK_ROWS)], gsem).wait()

        rowbase = lax.rem(t, 3) * _CHUNK_ROWS
        roi = t // 7
        ck = t - 7 * roi
        obase = (roi & 1) * (_BINS * _C)

        @pl.when((ck == 0) & (roi >= 2) & (tbase + roi - 2 < _NROI))
        def _drain_out():
            pltpu.make_async_copy(
                o_buf.at[pl.ds(0, _BINS * _C)],
                out_h.at[tbase], out_sem).wait()

        def cbin_body(bl, _c, roi=roi, ck=ck, rowbase=rowbase, obase=obase):
            g = ck * _CHUNK_BINS + bl
            wv = w_b[pl.ds((roi * _BINS + g) * _TPB, 16)]
            acc = [jnp.zeros((16,), jnp.float32) for _ in range(2 * (_W32 // 16))]
            for tp in range(_TPB):
                wt = _dg(wv, _splat_i32(tp))
                row = rowbase + bl * _TPB + tp
                for cg in range(_W32 // 16):
                    w32 = g_ring[row, pl.ds(cg * 16, 16)]
                    flo = lax.bitcast_convert_type(
                        lax.shift_left(w32, 16), jnp.float32)
                    fhi = lax.bitcast_convert_type(w32, jnp.float32)
                    acc[2 * cg] = acc[2 * cg] + wt * flo
                    acc[2 * cg + 1] = acc[2 * cg + 1] + wt * fhi
            for cg in range(_W32 // 16):
                o_buf[pl.ds(obase + g * _C + cg * 16, 16)] = acc[2 * cg]
                o_buf[pl.ds(obase + g * _C + 128 + cg * 16, 16)] = acc[2 * cg + 1]
            return _c

        lax.fori_loop(0, _CHUNK_BINS, cbin_body, 0)

        gro = tbase + roi

        @pl.when((ck == 6) & (gro < _NROI))
        def _flush():
            pltpu.async_copy(o_buf.at[pl.ds(obase, _BINS * _C)],
                             out_h.at[gro], out_sem)
        return _

    lax.fori_loop(0, _NCHUNKS, chunk_body, 0)
    n_out = jnp.minimum(jnp.maximum(_NROI - tbase, 0), _RPT)
    n_pending = jnp.maximum(n_out - (_RPT - 2), 0)

    def drain_body(i, _):
        pltpu.make_async_copy(o_buf.at[pl.ds(0, _BINS * _C)],
                              out_h.at[tbase], out_sem).wait()
        return _

    lax.fori_loop(0, n_pending, drain_body, 0)


@jax.jit
def _run(bidx, x1, y1, x2, y2, table):
    mesh = plsc.VectorSubcoreMesh(core_axis_name="c", subcore_axis_name="s")
    f = functools.partial(
        pl.kernel,
        out_type=jax.ShapeDtypeStruct((_NROI, _BINS * _C), jnp.float32),
        mesh=mesh,
        scratch_types=[
            pltpu.VMEM((_RPT,), jnp.float32),
            pltpu.VMEM((_RPT,), jnp.float32),
            pltpu.VMEM((_RPT,), jnp.float32),
            pltpu.VMEM((_RPT,), jnp.float32),
            pltpu.VMEM((_RPT,), jnp.float32),
            pltpu.VMEM((_RPT * _NTAP,), jnp.int32),
            pltpu.VMEM((_RPT * _NTAP,), jnp.float32),
            pltpu.VMEM((3 * _CHUNK_ROWS, _W32), jnp.int32),
            pltpu.VMEM((2 * _BINS * _C,), jnp.float32),
            pltpu.SemaphoreType.DMA,
            pltpu.SemaphoreType.DMA,
        ],
    )(_sc_body)
    return f(bidx, x1, y1, x2, y2, table)


def kernel(feat0, feat1, feat2, feat3, rois):
    feats = [feat0, feat1, feat2, feat3]
    c = feat0.shape[1]
    nrows = _LVL_OFF[3] + 2 * 16 * 16
    table = jnp.zeros((nrows, c // 2), jnp.int32)
    for f, off in zip(feats, _LVL_OFF):
        t = jnp.transpose(f.astype(jnp.bfloat16), (0, 2, 3, 1)).reshape(-1, c)
        lo = lax.bitcast_convert_type(t[:, :c // 2], jnp.uint16).astype(jnp.uint32)
        hi = lax.bitcast_convert_type(t[:, c // 2:], jnp.uint16).astype(jnp.uint32)
        table = lax.dynamic_update_slice(
            table, lax.bitcast_convert_type(lo | (hi << 16), jnp.int32), (off, 0))
    n = rois.shape[0]
    pad = jnp.broadcast_to(
        jnp.array([0.0, 0.0, 0.0, 10.0, 10.0], jnp.float32), (_NPAD - n, 5))
    rp = jnp.concatenate([rois, pad], axis=0)
    out = _run(rp[:, 0], rp[:, 1], rp[:, 2], rp[:, 3], rp[:, 4], table)
    return out.reshape(n, _OUT, _OUT, c).transpose(0, 3, 1, 2)
